# baseline (device time: 208633 ns/iter reference)
import jax
import jax.numpy as jnp
from jax import lax
from jax.experimental import pallas as pl
from jax.experimental.pallas import tpu as pltpu

N_DEV = 16


def kernel(partial, gamma):
    _, m_total, d = partial.shape
    m_per = m_total // N_DEV
    gamma2 = gamma.reshape(1, d)

    def body(x_ref, g_ref, out_ref, send_buf, comm_ref, send_sems, recv_sems):
        me = lax.axis_index("i")
        left = lax.rem(me - 1 + N_DEV, N_DEV)
        right = lax.rem(me + 1, N_DEV)

        barrier_sem = pltpu.get_barrier_semaphore()
        for nbr in (left, right):
            pl.semaphore_signal(
                barrier_sem, inc=1,
                device_id=(nbr,), device_id_type=pl.DeviceIdType.MESH,
            )
        pl.semaphore_wait(barrier_sem, 2)

        def chunk(c):
            return x_ref[0, pl.ds(c * m_per, m_per), :]

        for s in range(N_DEV - 1):
            c = lax.rem(me - (s + 1) + N_DEV, N_DEV)
            if s == 0:
                send_buf[:, :] = chunk(c)
            else:
                send_buf[:, :] = comm_ref[s - 1] + chunk(c)
            rdma = pltpu.make_async_remote_copy(
                src_ref=send_buf,
                dst_ref=comm_ref.at[s],
                send_sem=send_sems.at[s],
                recv_sem=recv_sems.at[s],
                device_id=(right,),
                device_id_type=pl.DeviceIdType.MESH,
            )
            rdma.start()
            rdma.wait()

        acc = comm_ref[N_DEV - 2] + chunk(me)
        ms = jnp.mean(acc * acc, axis=-1, keepdims=True)
        out_ref[:, :] = acc * lax.rsqrt(ms + 1e-6) * g_ref[:, :]

    return pl.pallas_call(
        body,
        out_shape=jax.ShapeDtypeStruct((m_per, d), jnp.float32),
        in_specs=[
            pl.BlockSpec(memory_space=pltpu.VMEM),
            pl.BlockSpec(memory_space=pltpu.VMEM),
        ],
        out_specs=pl.BlockSpec(memory_space=pltpu.VMEM),
        scratch_shapes=[
            pltpu.VMEM((m_per, d), jnp.float32),
            pltpu.VMEM((N_DEV - 1, m_per, d), jnp.float32),
            pltpu.SemaphoreType.DMA((N_DEV - 1,)),
            pltpu.SemaphoreType.DMA((N_DEV - 1,)),
        ],
        compiler_params=pltpu.CompilerParams(collective_id=0),
    )(partial, gamma2)


# device time: 130325 ns/iter; 1.6009x vs baseline; 1.6009x over previous
import jax
import jax.numpy as jnp
from jax import lax
from jax.experimental import pallas as pl
from jax.experimental.pallas import tpu as pltpu

N_DEV = 16

PERM = [0, 4, 8, 12, 13, 9, 5, 1, 2, 6, 10, 14, 15, 11, 7, 3]
POS = [0] * N_DEV
for _k, _v in enumerate(PERM):
    POS[_v] = _k


def kernel(partial, gamma):
    _, m_total, d = partial.shape
    m_per = m_total // N_DEV
    dh = d // 2
    gamma2 = gamma.reshape(1, d)

    idx = lax.axis_index("i")
    perm = jnp.array(PERM, dtype=jnp.int32)
    pos = jnp.array(POS, dtype=jnp.int32)
    p = pos[idx]
    s_arr = jnp.arange(N_DEV, dtype=jnp.int32)
    chunks_r = perm[(p - 1 - s_arr) % N_DEV]
    chunks_l = perm[(p + 1 + s_arr) % N_DEV]
    nbrs = jnp.stack([perm[(p - 1) % N_DEV], perm[(p + 1) % N_DEV]])

    def body(nbr_ref, cr_ref, cl_ref, x_ref, g_ref, out_ref,
             send_r, send_l, comm_r, comm_l,
             send_sems_r, recv_sems_r, send_sems_l, recv_sems_l):
        left = nbr_ref[0]
        right = nbr_ref[1]

        barrier_sem = pltpu.get_barrier_semaphore()
        for nbr in (left, right):
            pl.semaphore_signal(
                barrier_sem, inc=1,
                device_id=(nbr,), device_id_type=pl.DeviceIdType.MESH,
            )
        pl.semaphore_wait(barrier_sem, 2)

        def chunk_r(c):
            return x_ref[0, pl.ds(c * m_per, m_per), 0:dh]

        def chunk_l(c):
            return x_ref[0, pl.ds(c * m_per, m_per), dh:d]

        for s in range(N_DEV - 1):
            if s == 0:
                send_r[:, :] = chunk_r(cr_ref[0])
                send_l[:, :] = chunk_l(cl_ref[0])
            else:
                send_r[:, :] = comm_r[s - 1] + chunk_r(cr_ref[s])
                send_l[:, :] = comm_l[s - 1] + chunk_l(cl_ref[s])
            rdma_r = pltpu.make_async_remote_copy(
                src_ref=send_r,
                dst_ref=comm_r.at[s],
                send_sem=send_sems_r.at[s],
                recv_sem=recv_sems_r.at[s],
                device_id=(right,),
                device_id_type=pl.DeviceIdType.MESH,
            )
            rdma_l = pltpu.make_async_remote_copy(
                src_ref=send_l,
                dst_ref=comm_l.at[s],
                send_sem=send_sems_l.at[s],
                recv_sem=recv_sems_l.at[s],
                device_id=(left,),
                device_id_type=pl.DeviceIdType.MESH,
            )
            rdma_r.start()
            rdma_l.start()
            rdma_r.wait()
            rdma_l.wait()

        acc_r = comm_r[N_DEV - 2] + chunk_r(cr_ref[N_DEV - 1])
        acc_l = comm_l[N_DEV - 2] + chunk_l(cl_ref[N_DEV - 1])
        ssq = (jnp.sum(acc_r * acc_r, axis=-1, keepdims=True)
               + jnp.sum(acc_l * acc_l, axis=-1, keepdims=True))
        inv = lax.rsqrt(ssq / d + 1e-6)
        out_ref[:, 0:dh] = acc_r * inv * g_ref[:, 0:dh]
        out_ref[:, dh:d] = acc_l * inv * g_ref[:, dh:d]

    return pl.pallas_call(
        body,
        out_shape=jax.ShapeDtypeStruct((m_per, d), jnp.float32),
        in_specs=[
            pl.BlockSpec(memory_space=pltpu.SMEM),
            pl.BlockSpec(memory_space=pltpu.SMEM),
            pl.BlockSpec(memory_space=pltpu.SMEM),
            pl.BlockSpec(memory_space=pltpu.VMEM),
            pl.BlockSpec(memory_space=pltpu.VMEM),
        ],
        out_specs=pl.BlockSpec(memory_space=pltpu.VMEM),
        scratch_shapes=[
            pltpu.VMEM((m_per, dh), jnp.float32),
            pltpu.VMEM((m_per, dh), jnp.float32),
            pltpu.VMEM((N_DEV - 1, m_per, dh), jnp.float32),
            pltpu.VMEM((N_DEV - 1, m_per, dh), jnp.float32),
            pltpu.SemaphoreType.DMA((N_DEV - 1,)),
            pltpu.SemaphoreType.DMA((N_DEV - 1,)),
            pltpu.SemaphoreType.DMA((N_DEV - 1,)),
            pltpu.SemaphoreType.DMA((N_DEV - 1,)),
        ],
        compiler_params=pltpu.CompilerParams(collective_id=0),
    )(nbrs, chunks_r, chunks_l, partial, gamma2)


# device time: 104025 ns/iter; 2.0056x vs baseline; 1.2528x over previous
import jax
import jax.numpy as jnp
from jax import lax
from jax.experimental import pallas as pl
from jax.experimental.pallas import tpu as pltpu

N_DEV = 16
B = 2

PERM = [0, 4, 8, 12, 13, 9, 5, 1, 2, 6, 10, 14, 15, 11, 7, 3]
POS = [0] * N_DEV
for _k, _v in enumerate(PERM):
    POS[_v] = _k


def kernel(partial, gamma):
    _, m_total, d = partial.shape
    m_per = m_total // N_DEV
    m_sub = m_per // B
    dh = d // 2
    gamma2 = gamma.reshape(1, d)

    idx = lax.axis_index("i")
    perm = jnp.array(PERM, dtype=jnp.int32)
    pos = jnp.array(POS, dtype=jnp.int32)
    p = pos[idx]
    s_arr = jnp.arange(N_DEV, dtype=jnp.int32)
    chunks_r = perm[(p - 1 - s_arr) % N_DEV]
    chunks_l = perm[(p + 1 + s_arr) % N_DEV]
    nbrs = jnp.stack([perm[(p - 1) % N_DEV], perm[(p + 1) % N_DEV]])

    def body(nbr_ref, cr_ref, cl_ref, x_ref, g_ref, out_ref,
             send_r, send_l, comm_r, comm_l,
             send_sems_r, recv_sems_r, send_sems_l, recv_sems_l):
        left = nbr_ref[0]
        right = nbr_ref[1]

        barrier_sem = pltpu.get_barrier_semaphore()
        for nbr in (left, right):
            pl.semaphore_signal(
                barrier_sem, inc=1,
                device_id=(nbr,), device_id_type=pl.DeviceIdType.MESH,
            )
        pl.semaphore_wait(barrier_sem, 2)

        def sub_r(c, b):
            return x_ref[0, pl.ds(c * m_per + b * m_sub, m_sub), 0:dh]

        def sub_l(c, b):
            return x_ref[0, pl.ds(c * m_per + b * m_sub, m_sub), dh:d]

        def make(dirn, s, b):
            if dirn == 0:
                return pltpu.make_async_remote_copy(
                    src_ref=send_r.at[b],
                    dst_ref=comm_r.at[s, pl.ds(b * m_sub, m_sub), :],
                    send_sem=send_sems_r.at[s, b],
                    recv_sem=recv_sems_r.at[s, b],
                    device_id=(right,),
                    device_id_type=pl.DeviceIdType.MESH,
                )
            return pltpu.make_async_remote_copy(
                src_ref=send_l.at[b],
                dst_ref=comm_l.at[s, pl.ds(b * m_sub, m_sub), :],
                send_sem=send_sems_l.at[s, b],
                recv_sem=recv_sems_l.at[s, b],
                device_id=(left,),
                device_id_type=pl.DeviceIdType.MESH,
            )

        rdmas = {}

        for b in range(B):
            send_r[b] = sub_r(cr_ref[0], b)
            send_l[b] = sub_l(cl_ref[0], b)
            rdmas[(0, 0, b)] = make(0, 0, b)
            rdmas[(1, 0, b)] = make(1, 0, b)
            rdmas[(0, 0, b)].start()
            rdmas[(1, 0, b)].start()

        for s in range(1, N_DEV - 1):
            for b in range(B):
                rdmas[(0, s - 1, b)].wait_send()
                rdmas[(0, s - 1, b)].wait_recv()
                send_r[b] = (comm_r[s - 1, pl.ds(b * m_sub, m_sub), :]
                             + sub_r(cr_ref[s], b))
                rdmas[(0, s, b)] = make(0, s, b)
                rdmas[(0, s, b)].start()

                rdmas[(1, s - 1, b)].wait_send()
                rdmas[(1, s - 1, b)].wait_recv()
                send_l[b] = (comm_l[s - 1, pl.ds(b * m_sub, m_sub), :]
                             + sub_l(cl_ref[s], b))
                rdmas[(1, s, b)] = make(1, s, b)
                rdmas[(1, s, b)].start()

        for b in range(B):
            rdmas[(0, N_DEV - 2, b)].wait_send()
            rdmas[(1, N_DEV - 2, b)].wait_send()
            rdmas[(0, N_DEV - 2, b)].wait_recv()
            rdmas[(1, N_DEV - 2, b)].wait_recv()

        acc_r = comm_r[N_DEV - 2] + x_ref[0, pl.ds(cr_ref[N_DEV - 1] * m_per, m_per), 0:dh]
        acc_l = comm_l[N_DEV - 2] + x_ref[0, pl.ds(cl_ref[N_DEV - 1] * m_per, m_per), dh:d]
        ssq = (jnp.sum(acc_r * acc_r, axis=-1, keepdims=True)
               + jnp.sum(acc_l * acc_l, axis=-1, keepdims=True))
        inv = lax.rsqrt(ssq / d + 1e-6)
        out_ref[:, 0:dh] = acc_r * inv * g_ref[:, 0:dh]
        out_ref[:, dh:d] = acc_l * inv * g_ref[:, dh:d]

    return pl.pallas_call(
        body,
        out_shape=jax.ShapeDtypeStruct((m_per, d), jnp.float32),
        in_specs=[
            pl.BlockSpec(memory_space=pltpu.SMEM),
            pl.BlockSpec(memory_space=pltpu.SMEM),
            pl.BlockSpec(memory_space=pltpu.SMEM),
            pl.BlockSpec(memory_space=pltpu.VMEM),
            pl.BlockSpec(memory_space=pltpu.VMEM),
        ],
        out_specs=pl.BlockSpec(memory_space=pltpu.VMEM),
        scratch_shapes=[
            pltpu.VMEM((B, m_sub, dh), jnp.float32),
            pltpu.VMEM((B, m_sub, dh), jnp.float32),
            pltpu.VMEM((N_DEV - 1, m_per, dh), jnp.float32),
            pltpu.VMEM((N_DEV - 1, m_per, dh), jnp.float32),
            pltpu.SemaphoreType.DMA((N_DEV - 1, B)),
            pltpu.SemaphoreType.DMA((N_DEV - 1, B)),
            pltpu.SemaphoreType.DMA((N_DEV - 1, B)),
            pltpu.SemaphoreType.DMA((N_DEV - 1, B)),
        ],
        compiler_params=pltpu.CompilerParams(collective_id=0),
    )(nbrs, chunks_r, chunks_l, partial, gamma2)


# device time: 97727 ns/iter; 2.1349x vs baseline; 1.0644x over previous
import jax
import jax.numpy as jnp
from jax import lax
from jax.experimental import pallas as pl
from jax.experimental.pallas import tpu as pltpu

N_DEV = 16
B = 2


def _perm_of(k):
    q = k // 4
    v01 = jnp.where(q == 0, 4 * k, 29 - 4 * k)
    v23 = jnp.where(q == 2, 4 * k - 30, 63 - 4 * k)
    return jnp.where(q < 2, v01, v23)


def _pos_of(me):
    r = me % 4
    v01 = jnp.where(r == 0, me // 4, (29 - me) // 4)
    v23 = jnp.where(r == 2, (me + 30) // 4, (63 - me) // 4)
    return jnp.where(r < 2, v01, v23)


def kernel(partial, gamma):
    _, m_total, d = partial.shape
    m_per = m_total // N_DEV
    m_sub = m_per // B
    dh = d // 2
    gamma2 = gamma.reshape(1, d)

    def body(x_ref, g_ref, out_ref,
             send_r, send_l, comm_r, comm_l,
             send_sems_r, recv_sems_r, send_sems_l, recv_sems_l):
        me = lax.axis_index("i")
        p = _pos_of(me)
        left = _perm_of((p + N_DEV - 1) % N_DEV)
        right = _perm_of((p + 1) % N_DEV)
        cr = [_perm_of((p + (2 * N_DEV - 1 - s)) % N_DEV) for s in range(N_DEV)]
        cl = [_perm_of((p + (1 + s)) % N_DEV) for s in range(N_DEV)]

        barrier_sem = pltpu.get_barrier_semaphore()
        for nbr in (left, right):
            pl.semaphore_signal(
                barrier_sem, inc=1,
                device_id=(nbr,), device_id_type=pl.DeviceIdType.MESH,
            )
        pl.semaphore_wait(barrier_sem, 2)

        def sub_r(c, b):
            return x_ref[0, pl.ds(c * m_per + b * m_sub, m_sub), 0:dh]

        def sub_l(c, b):
            return x_ref[0, pl.ds(c * m_per + b * m_sub, m_sub), dh:d]

        def make(dirn, s, b):
            if dirn == 0:
                return pltpu.make_async_remote_copy(
                    src_ref=send_r.at[b],
                    dst_ref=comm_r.at[s, pl.ds(b * m_sub, m_sub), :],
                    send_sem=send_sems_r.at[s, b],
                    recv_sem=recv_sems_r.at[s, b],
                    device_id=(right,),
                    device_id_type=pl.DeviceIdType.MESH,
                )
            return pltpu.make_async_remote_copy(
                src_ref=send_l.at[b],
                dst_ref=comm_l.at[s, pl.ds(b * m_sub, m_sub), :],
                send_sem=send_sems_l.at[s, b],
                recv_sem=recv_sems_l.at[s, b],
                device_id=(left,),
                device_id_type=pl.DeviceIdType.MESH,
            )

        rdmas = {}

        for b in range(B):
            send_r[b] = sub_r(cr[0], b)
            send_l[b] = sub_l(cl[0], b)
            rdmas[(0, 0, b)] = make(0, 0, b)
            rdmas[(1, 0, b)] = make(1, 0, b)
            rdmas[(0, 0, b)].start()
            rdmas[(1, 0, b)].start()

        for s in range(1, N_DEV - 1):
            for b in range(B):
                rdmas[(0, s - 1, b)].wait_send()
                rdmas[(0, s - 1, b)].wait_recv()
                send_r[b] = (comm_r[s - 1, pl.ds(b * m_sub, m_sub), :]
                             + sub_r(cr[s], b))
                rdmas[(0, s, b)] = make(0, s, b)
                rdmas[(0, s, b)].start()

                rdmas[(1, s - 1, b)].wait_send()
                rdmas[(1, s - 1, b)].wait_recv()
                send_l[b] = (comm_l[s - 1, pl.ds(b * m_sub, m_sub), :]
                             + sub_l(cl[s], b))
                rdmas[(1, s, b)] = make(1, s, b)
                rdmas[(1, s, b)].start()

        for b in range(B):
            rows = pl.ds(b * m_sub, m_sub)
            rdmas[(0, N_DEV - 2, b)].wait_recv()
            rdmas[(1, N_DEV - 2, b)].wait_recv()
            acc_r = comm_r[N_DEV - 2, rows, :] + sub_r(cr[N_DEV - 1], b)
            acc_l = comm_l[N_DEV - 2, rows, :] + sub_l(cl[N_DEV - 1], b)
            ssq = (jnp.sum(acc_r * acc_r, axis=-1, keepdims=True)
                   + jnp.sum(acc_l * acc_l, axis=-1, keepdims=True))
            inv = lax.rsqrt(ssq / d + 1e-6)
            out_ref[rows, 0:dh] = acc_r * inv * g_ref[:, 0:dh]
            out_ref[rows, dh:d] = acc_l * inv * g_ref[:, dh:d]

        for b in range(B):
            rdmas[(0, N_DEV - 2, b)].wait_send()
            rdmas[(1, N_DEV - 2, b)].wait_send()

    return pl.pallas_call(
        body,
        out_shape=jax.ShapeDtypeStruct((m_per, d), jnp.float32),
        in_specs=[
            pl.BlockSpec(memory_space=pltpu.VMEM),
            pl.BlockSpec(memory_space=pltpu.VMEM),
        ],
        out_specs=pl.BlockSpec(memory_space=pltpu.VMEM),
        scratch_shapes=[
            pltpu.VMEM((B, m_sub, dh), jnp.float32),
            pltpu.VMEM((B, m_sub, dh), jnp.float32),
            pltpu.VMEM((N_DEV - 1, m_per, dh), jnp.float32),
            pltpu.VMEM((N_DEV - 1, m_per, dh), jnp.float32),
            pltpu.SemaphoreType.DMA((N_DEV - 1, B)),
            pltpu.SemaphoreType.DMA((N_DEV - 1, B)),
            pltpu.SemaphoreType.DMA((N_DEV - 1, B)),
            pltpu.SemaphoreType.DMA((N_DEV - 1, B)),
        ],
        compiler_params=pltpu.CompilerParams(collective_id=0),
    )(partial, gamma2)
